# Initial kernel scaffold; baseline (speedup 1.0000x reference)
import jax
import jax.numpy as jnp
from jax.experimental import pallas as pl


def kernel(time_segs, edges, W_edge, b_edge, W1, b1, W_d, b_d):
    raise NotImplementedError("write your pallas kernel here")


# speedup vs baseline: 5.2025x; 5.2025x over previous
import functools

import jax
import jax.numpy as jnp
from jax import lax
from jax.experimental import pallas as pl
from jax.experimental.pallas import tpu as pltpu
from jax.experimental.pallas import tpu_sc as plsc

NN = 10000
NE = 320000
ND = 128
HE = 64
HN = 64
PRED = 2

NP_ = 10240
RB = 1024
C = 128
NC, NS = 2, 16
NW = NC * NS
CPW = 80
NEP = NW * CPW * C
RT = NP_ // NS

_f32 = jnp.float32


def _pq_body(x_ref, wes_ref, wed_ref, be_ref, p_ref, q_ref):
    x = x_ref[...]
    p_ref[...] = jnp.dot(x, wes_ref[...], preferred_element_type=_f32)
    q_ref[...] = jnp.dot(x, wed_ref[...], preferred_element_type=_f32) + be_ref[...]


def _step_body(x_ref, agg_ref, w1a_ref, w1b_ref, b1_ref, wd_ref, bd_ref,
               wes_ref, wed_ref, be_ref, nxt_ref, p_ref, q_ref):
    x = x_ref[...]
    agg = agg_ref[0] + agg_ref[1]
    h = jnp.maximum(
        jnp.dot(x, w1a_ref[...], preferred_element_type=_f32)
        + jnp.dot(agg, w1b_ref[...], preferred_element_type=_f32)
        + b1_ref[...], 0.0)
    nxt = x + jnp.dot(h, wd_ref[...], preferred_element_type=_f32) + bd_ref[...]
    nxt_ref[...] = nxt
    p_ref[...] = jnp.dot(nxt, wes_ref[...], preferred_element_type=_f32)
    q_ref[...] = jnp.dot(nxt, wed_ref[...], preferred_element_type=_f32) + be_ref[...]


def _final_body(x_ref, agg_ref, w1a_ref, w1b_ref, b1_ref, wd_ref, bd_ref,
                nxt_ref):
    x = x_ref[...]
    agg = agg_ref[0] + agg_ref[1]
    h = jnp.maximum(
        jnp.dot(x, w1a_ref[...], preferred_element_type=_f32)
        + jnp.dot(agg, w1b_ref[...], preferred_element_type=_f32)
        + b1_ref[...], 0.0)
    nxt_ref[...] = x + jnp.dot(h, wd_ref[...], preferred_element_type=_f32) + bd_ref[...]


def _full(shape):
    return pl.BlockSpec(shape, lambda i: tuple(0 for _ in shape))


def _rows(shape):
    return pl.BlockSpec(shape, lambda i: (i,) + tuple(0 for _ in shape[1:]))


_GRID = NP_ // RB

_pq_call = pl.pallas_call(
    _pq_body,
    grid=(_GRID,),
    in_specs=[_rows((RB, ND)), _full((ND, HE)), _full((ND, HE)), _full((1, HE))],
    out_specs=[_rows((RB, HE)), _rows((RB, HE))],
    out_shape=[jax.ShapeDtypeStruct((NP_, HE), _f32)] * 2,
)

_step_call = pl.pallas_call(
    _step_body,
    grid=(_GRID,),
    in_specs=[
        _rows((RB, ND)),
        pl.BlockSpec((NC, RB, HE), lambda i: (0, i, 0)),
        _full((ND, HN)), _full((HE, HN)), _full((1, HN)),
        _full((HN, ND)), _full((1, ND)),
        _full((ND, HE)), _full((ND, HE)), _full((1, HE)),
    ],
    out_specs=[_rows((RB, ND)), _rows((RB, HE)), _rows((RB, HE))],
    out_shape=[jax.ShapeDtypeStruct((NP_, ND), _f32),
               jax.ShapeDtypeStruct((NP_, HE), _f32),
               jax.ShapeDtypeStruct((NP_, HE), _f32)],
)

_final_call = pl.pallas_call(
    _final_body,
    grid=(_GRID,),
    in_specs=[
        _rows((RB, ND)),
        pl.BlockSpec((NC, RB, HE), lambda i: (0, i, 0)),
        _full((ND, HN)), _full((HE, HN)), _full((1, HN)),
        _full((HN, ND)), _full((1, ND)),
    ],
    out_specs=_rows((RB, ND)),
    out_shape=jax.ShapeDtypeStruct((NP_, ND), _f32),
)


_mesh = plsc.VectorSubcoreMesh(core_axis_name="c", subcore_axis_name="s")


@functools.partial(
    pl.kernel,
    out_type=jax.ShapeDtypeStruct((NC, NP_, HE), _f32),
    mesh=_mesh,
    compiler_params=pltpu.CompilerParams(use_tc_tiling_on_sc=False),
    scratch_types=[
        pltpu.VMEM((CPW, C), jnp.int32),
        pltpu.VMEM((CPW, C), jnp.int32),
        pltpu.VMEM((C, HE), _f32),
        pltpu.VMEM((C, HE), _f32),
        pltpu.VMEM((RT, HE), _f32),
        pltpu.VMEM_SHARED((NP_, HE), _f32),
        pltpu.SemaphoreType.DMA,
        pltpu.SemaphoreType.DMA,
    ],
)
def _edge_agg(p_hbm, q_hbm, srcb_hbm, dstb_hbm, out_hbm,
              src_v, dst_v, pbuf, qbuf, zbuf, agg_sh, sem_p, sem_q):
    cid = lax.axis_index("c")
    sid = lax.axis_index("s")
    wid = sid * NC + cid
    row0 = sid * RT

    def _zero_row(r, _):
        for c4 in range(HE // 16):
            zbuf[r, pl.ds(c4 * 16, 16)] = jnp.zeros((16,), _f32)
        return 0

    lax.fori_loop(0, RT, _zero_row, 0)
    pltpu.sync_copy(zbuf, agg_sh.at[pl.ds(row0, RT)])

    pltpu.sync_copy(srcb_hbm.at[pl.ds(wid * CPW, CPW)], src_v)
    pltpu.sync_copy(dstb_hbm.at[pl.ds(wid * CPW, CPW)], dst_v)
    plsc.subcore_barrier()

    def _chunk(j, _):
        cp = pltpu.async_copy(p_hbm.at[src_v.at[j]], pbuf, sem_p)
        cq = pltpu.async_copy(q_hbm.at[dst_v.at[j]], qbuf, sem_q)
        cp.wait()
        cq.wait()

        def _relu_row(r, _):
            for c4 in range(HE // 16):
                s = pl.ds(c4 * 16, 16)
                pbuf[r, s] = jnp.maximum(pbuf[r, s] + qbuf[r, s], 0.0)
            return 0

        lax.fori_loop(0, C, _relu_row, 0)
        pltpu.sync_copy(pbuf, agg_sh.at[dst_v.at[j]], add=True)
        return 0

    lax.fori_loop(0, CPW, _chunk, 0)
    plsc.subcore_barrier()

    pltpu.sync_copy(agg_sh.at[pl.ds(row0, RT)], zbuf)
    pltpu.sync_copy(zbuf, out_hbm.at[cid, pl.ds(row0, RT)])


def kernel(time_segs, edges, W_edge, b_edge, W1, b1, W_d, b_d):
    x0 = time_segs[0, 0]
    x = jnp.zeros((NP_, ND), _f32).at[:NN].set(x0)

    wes = W_edge[:ND]
    wed = W_edge[ND:]
    be = b_edge.reshape(1, HE)
    w1a = W1[:ND]
    w1b = W1[ND:]
    b1r = b1.reshape(1, HN)
    bdr = b_d.reshape(1, ND)

    pad = jnp.full((NEP - NE,), NN, jnp.int32)
    srcb = jnp.concatenate([edges[0], pad]).reshape(NW * CPW, C)
    dstb = jnp.concatenate([edges[1], pad]).reshape(NW * CPW, C)

    p, q = _pq_call(x, wes, wed, be)
    outs = []
    for step in range(PRED):
        aggp = _edge_agg(p, q, srcb, dstb)
        if step + 1 < PRED:
            x, p, q = _step_call(x, aggp, w1a, w1b, b1r, W_d, bdr, wes, wed, be)
        else:
            x = _final_call(x, aggp, w1a, w1b, b1r, W_d, bdr)
        outs.append(x[:NN])

    return jnp.stack(outs)[None]


# speedup vs baseline: 6.4984x; 1.2491x over previous
import functools

import jax
import jax.numpy as jnp
from jax import lax
from jax.experimental import pallas as pl
from jax.experimental.pallas import tpu as pltpu
from jax.experimental.pallas import tpu_sc as plsc

NN = 10000
NE = 320000
ND = 128
HE = 64
HN = 64
PRED = 2

NP_ = 10240
RB = 1024
C = 128
NC, NS = 2, 16
NW = NC * NS
CPW = 80
NEP = NW * CPW * C
RT = NP_ // NS

_f32 = jnp.float32


def _pq_body(x_ref, wes_ref, wed_ref, be_ref, p_ref, q_ref):
    x = x_ref[...]
    p_ref[...] = jnp.dot(x, wes_ref[...], preferred_element_type=_f32)
    q_ref[...] = jnp.dot(x, wed_ref[...], preferred_element_type=_f32) + be_ref[...]


def _step_body(x_ref, agg_ref, w1a_ref, w1b_ref, b1_ref, wd_ref, bd_ref,
               wes_ref, wed_ref, be_ref, nxt_ref, p_ref, q_ref):
    x = x_ref[...]
    agg = agg_ref[0] + agg_ref[1]
    h = jnp.maximum(
        jnp.dot(x, w1a_ref[...], preferred_element_type=_f32)
        + jnp.dot(agg, w1b_ref[...], preferred_element_type=_f32)
        + b1_ref[...], 0.0)
    nxt = x + jnp.dot(h, wd_ref[...], preferred_element_type=_f32) + bd_ref[...]
    nxt_ref[...] = nxt
    p_ref[...] = jnp.dot(nxt, wes_ref[...], preferred_element_type=_f32)
    q_ref[...] = jnp.dot(nxt, wed_ref[...], preferred_element_type=_f32) + be_ref[...]


def _final_body(x_ref, agg_ref, w1a_ref, w1b_ref, b1_ref, wd_ref, bd_ref,
                nxt_ref):
    x = x_ref[...]
    agg = agg_ref[0] + agg_ref[1]
    h = jnp.maximum(
        jnp.dot(x, w1a_ref[...], preferred_element_type=_f32)
        + jnp.dot(agg, w1b_ref[...], preferred_element_type=_f32)
        + b1_ref[...], 0.0)
    nxt_ref[...] = x + jnp.dot(h, wd_ref[...], preferred_element_type=_f32) + bd_ref[...]


def _full(shape):
    return pl.BlockSpec(shape, lambda i: tuple(0 for _ in shape))


def _rows(shape):
    return pl.BlockSpec(shape, lambda i: (i,) + tuple(0 for _ in shape[1:]))


_GRID = NP_ // RB

_pq_call = pl.pallas_call(
    _pq_body,
    grid=(_GRID,),
    in_specs=[_rows((RB, ND)), _full((ND, HE)), _full((ND, HE)), _full((1, HE))],
    out_specs=[_rows((RB, HE)), _rows((RB, HE))],
    out_shape=[jax.ShapeDtypeStruct((NP_, HE), _f32)] * 2,
)

_step_call = pl.pallas_call(
    _step_body,
    grid=(_GRID,),
    in_specs=[
        _rows((RB, ND)),
        pl.BlockSpec((NC, RB, HE), lambda i: (0, i, 0)),
        _full((ND, HN)), _full((HE, HN)), _full((1, HN)),
        _full((HN, ND)), _full((1, ND)),
        _full((ND, HE)), _full((ND, HE)), _full((1, HE)),
    ],
    out_specs=[_rows((RB, ND)), _rows((RB, HE)), _rows((RB, HE))],
    out_shape=[jax.ShapeDtypeStruct((NP_, ND), _f32),
               jax.ShapeDtypeStruct((NP_, HE), _f32),
               jax.ShapeDtypeStruct((NP_, HE), _f32)],
)

_final_call = pl.pallas_call(
    _final_body,
    grid=(_GRID,),
    in_specs=[
        _rows((RB, ND)),
        pl.BlockSpec((NC, RB, HE), lambda i: (0, i, 0)),
        _full((ND, HN)), _full((HE, HN)), _full((1, HN)),
        _full((HN, ND)), _full((1, ND)),
    ],
    out_specs=_rows((RB, ND)),
    out_shape=jax.ShapeDtypeStruct((NP_, ND), _f32),
)


_mesh = plsc.VectorSubcoreMesh(core_axis_name="c", subcore_axis_name="s")


@functools.partial(
    pl.kernel,
    out_type=jax.ShapeDtypeStruct((NC, NP_, HE), _f32),
    mesh=_mesh,
    compiler_params=pltpu.CompilerParams(use_tc_tiling_on_sc=False),
    scratch_types=[
        pltpu.VMEM((CPW, C), jnp.int32),
        pltpu.VMEM((CPW, C), jnp.int32),
        pltpu.VMEM((2, C, HE), _f32),
        pltpu.VMEM((2, C, HE), _f32),
        pltpu.VMEM((2, C, HE), _f32),
        pltpu.VMEM_SHARED((NP_, HE), _f32),
        pltpu.SemaphoreType.DMA,
        pltpu.SemaphoreType.DMA,
        pltpu.SemaphoreType.DMA,
        pltpu.SemaphoreType.DMA,
    ],
)
def _edge_agg(p_hbm, q_hbm, srcb_hbm, dstb_hbm, out_hbm,
              src_v, dst_v, pbuf, qbuf, mbuf, agg_sh, sp0, sp1, sq0, sq1):
    cid = lax.axis_index("c")
    sid = lax.axis_index("s")
    wid = sid * NC + cid
    row0 = sid * RT
    sems = ((sp0, sq0), (sp1, sq1))

    def _zero_row(r, _):
        for c4 in range(HE // 16):
            mbuf[0, r, pl.ds(c4 * 16, 16)] = jnp.zeros((16,), _f32)
        return 0

    lax.fori_loop(0, C, _zero_row, 0)
    for k in range(RT // C):
        pltpu.sync_copy(mbuf.at[0], agg_sh.at[pl.ds(row0 + k * C, C)])

    pltpu.sync_copy(srcb_hbm.at[pl.ds(wid * CPW, CPW)], src_v)
    pltpu.sync_copy(dstb_hbm.at[pl.ds(wid * CPW, CPW)], dst_v)
    plsc.subcore_barrier()

    def _fire(c, b):
        pltpu.async_copy(p_hbm.at[src_v.at[c]], pbuf.at[b], sems[b][0])
        pltpu.async_copy(q_hbm.at[dst_v.at[c]], qbuf.at[b], sems[b][1])

    _fire(0, 0)
    _fire(1, 1)

    def _pair(i, _):
        for b in range(2):
            c = 2 * i + b
            pltpu.make_async_copy(p_hbm.at[src_v.at[c]], pbuf.at[b], sems[b][0]).wait()
            pltpu.make_async_copy(q_hbm.at[dst_v.at[c]], qbuf.at[b], sems[b][1]).wait()

            def _relu_row(r, _):
                for c4 in range(HE // 16):
                    s = pl.ds(c4 * 16, 16)
                    mbuf[b, r, s] = jnp.maximum(pbuf[b, r, s] + qbuf[b, r, s], 0.0)
                return 0

            lax.fori_loop(0, C, _relu_row, 0)

            @pl.when(c + 2 < CPW)
            def _():
                _fire(c + 2, b)

            pltpu.sync_copy(mbuf.at[b], agg_sh.at[dst_v.at[c]], add=True)
        return 0

    lax.fori_loop(0, CPW // 2, _pair, 0)
    plsc.subcore_barrier()

    for k in range(RT // C):
        pltpu.sync_copy(agg_sh.at[pl.ds(row0 + k * C, C)], mbuf.at[0])
        pltpu.sync_copy(mbuf.at[0], out_hbm.at[cid, pl.ds(row0 + k * C, C)])


def kernel(time_segs, edges, W_edge, b_edge, W1, b1, W_d, b_d):
    x0 = time_segs[0, 0]
    x = jnp.zeros((NP_, ND), _f32).at[:NN].set(x0)

    wes = W_edge[:ND]
    wed = W_edge[ND:]
    be = b_edge.reshape(1, HE)
    w1a = W1[:ND]
    w1b = W1[ND:]
    b1r = b1.reshape(1, HN)
    bdr = b_d.reshape(1, ND)

    pad = jnp.full((NEP - NE,), NN, jnp.int32)
    srcb = jnp.concatenate([edges[0], pad]).reshape(NW * CPW, C)
    dstb = jnp.concatenate([edges[1], pad]).reshape(NW * CPW, C)

    p, q = _pq_call(x, wes, wed, be)
    outs = []
    for step in range(PRED):
        aggp = _edge_agg(p, q, srcb, dstb)
        if step + 1 < PRED:
            x, p, q = _step_call(x, aggp, w1a, w1b, b1r, W_d, bdr, wes, wed, be)
        else:
            x = _final_call(x, aggp, w1a, w1b, b1r, W_d, bdr)
        outs.append(x[:NN])

    return jnp.stack(outs)[None]


# speedup vs baseline: 8.6294x; 1.3279x over previous
import functools

import numpy as np

import jax
import jax.numpy as jnp
from jax import lax
from jax.experimental import pallas as pl
from jax.experimental.pallas import tpu as pltpu
from jax.experimental.pallas import tpu_sc as plsc

NN = 10000
NE = 320000
ND = 128
HE = 64
HN = 64
PRED = 2

NP_ = 10240
RB = 1024
C = 128
NC, NS = 2, 16
NW = NC * NS
CPW = 80
NEP = NW * CPW * C
RT = NP_ // NS

_f32 = jnp.float32
_bf16 = jnp.bfloat16

_TAU = np.concatenate([np.arange(0, 32, 2), np.arange(1, 32, 2),
                       np.arange(32, 64, 2), np.arange(33, 64, 2)])


def _pq_body(x_ref, wes_ref, wed_ref, be_ref, p_ref, q_ref):
    x = x_ref[...]
    p_ref[...] = jnp.dot(x, wes_ref[...], preferred_element_type=_f32).astype(_bf16)
    q_ref[...] = (jnp.dot(x, wed_ref[...], preferred_element_type=_f32)
                  + be_ref[...]).astype(_bf16)


def _step_body(x_ref, agg_ref, w1a_ref, w1b_ref, b1_ref, wd_ref, bd_ref,
               wes_ref, wed_ref, be_ref, nxt_ref, p_ref, q_ref):
    x = x_ref[...]
    agg = agg_ref[0] + agg_ref[1]
    h = jnp.maximum(
        jnp.dot(x, w1a_ref[...], preferred_element_type=_f32)
        + jnp.dot(agg, w1b_ref[...], preferred_element_type=_f32)
        + b1_ref[...], 0.0)
    nxt = x + jnp.dot(h, wd_ref[...], preferred_element_type=_f32) + bd_ref[...]
    nxt_ref[...] = nxt
    p_ref[...] = jnp.dot(nxt, wes_ref[...], preferred_element_type=_f32).astype(_bf16)
    q_ref[...] = (jnp.dot(nxt, wed_ref[...], preferred_element_type=_f32)
                  + be_ref[...]).astype(_bf16)


def _final_body(x_ref, agg_ref, w1a_ref, w1b_ref, b1_ref, wd_ref, bd_ref,
                nxt_ref):
    x = x_ref[...]
    agg = agg_ref[0] + agg_ref[1]
    h = jnp.maximum(
        jnp.dot(x, w1a_ref[...], preferred_element_type=_f32)
        + jnp.dot(agg, w1b_ref[...], preferred_element_type=_f32)
        + b1_ref[...], 0.0)
    nxt_ref[...] = x + jnp.dot(h, wd_ref[...], preferred_element_type=_f32) + bd_ref[...]


def _full(shape):
    return pl.BlockSpec(shape, lambda i: tuple(0 for _ in shape))


def _rows(shape):
    return pl.BlockSpec(shape, lambda i: (i,) + tuple(0 for _ in shape[1:]))


_GRID = NP_ // RB

_pq_call = pl.pallas_call(
    _pq_body,
    grid=(_GRID,),
    in_specs=[_rows((RB, ND)), _full((ND, HE)), _full((ND, HE)), _full((1, HE))],
    out_specs=[_rows((RB, HE)), _rows((RB, HE))],
    out_shape=[jax.ShapeDtypeStruct((NP_, HE), _bf16)] * 2,
)

_step_call = pl.pallas_call(
    _step_body,
    grid=(_GRID,),
    in_specs=[
        _rows((RB, ND)),
        pl.BlockSpec((NC, RB, HE), lambda i: (0, i, 0)),
        _full((ND, HN)), _full((HE, HN)), _full((1, HN)),
        _full((HN, ND)), _full((1, ND)),
        _full((ND, HE)), _full((ND, HE)), _full((1, HE)),
    ],
    out_specs=[_rows((RB, ND)), _rows((RB, HE)), _rows((RB, HE))],
    out_shape=[jax.ShapeDtypeStruct((NP_, ND), _f32),
               jax.ShapeDtypeStruct((NP_, HE), _bf16),
               jax.ShapeDtypeStruct((NP_, HE), _bf16)],
)

_final_call = pl.pallas_call(
    _final_body,
    grid=(_GRID,),
    in_specs=[
        _rows((RB, ND)),
        pl.BlockSpec((NC, RB, HE), lambda i: (0, i, 0)),
        _full((ND, HN)), _full((HE, HN)), _full((1, HN)),
        _full((HN, ND)), _full((1, ND)),
    ],
    out_specs=_rows((RB, ND)),
    out_shape=jax.ShapeDtypeStruct((NP_, ND), _f32),
)


_mesh = plsc.VectorSubcoreMesh(core_axis_name="c", subcore_axis_name="s")


@functools.partial(
    pl.kernel,
    out_type=jax.ShapeDtypeStruct((NC, NP_, HE), _f32),
    mesh=_mesh,
    compiler_params=pltpu.CompilerParams(use_tc_tiling_on_sc=False,
                                         needs_layout_passes=False),
    scratch_types=[
        pltpu.VMEM((CPW, C), jnp.int32),
        pltpu.VMEM((CPW, C), jnp.int32),
        pltpu.VMEM((2, C, HE), _bf16),
        pltpu.VMEM((2, C, HE), _bf16),
        pltpu.VMEM((2, C, HE), _f32),
        pltpu.VMEM_SHARED((NP_, HE), _f32),
        pltpu.SemaphoreType.DMA,
        pltpu.SemaphoreType.DMA,
        pltpu.SemaphoreType.DMA,
        pltpu.SemaphoreType.DMA,
    ],
)
def _edge_agg(p_hbm, q_hbm, srcb_hbm, dstb_hbm, out_hbm,
              src_v, dst_v, pbuf, qbuf, mbuf, agg_sh, sp0, sp1, sq0, sq1):
    cid = lax.axis_index("c")
    sid = lax.axis_index("s")
    wid = sid * NC + cid
    row0 = sid * RT
    sems = ((sp0, sq0), (sp1, sq1))

    def _zero_row(r, _):
        for c4 in range(HE // 16):
            mbuf[0, r, pl.ds(c4 * 16, 16)] = jnp.zeros((16,), _f32)
        return 0

    lax.fori_loop(0, C, _zero_row, 0)
    for k in range(RT // C):
        pltpu.sync_copy(mbuf.at[0], agg_sh.at[pl.ds(row0 + k * C, C)])

    pltpu.sync_copy(srcb_hbm.at[pl.ds(wid * CPW, CPW)], src_v)
    pltpu.sync_copy(dstb_hbm.at[pl.ds(wid * CPW, CPW)], dst_v)
    plsc.subcore_barrier()

    def _fire(c, b):
        pltpu.async_copy(p_hbm.at[src_v.at[c]], pbuf.at[b], sems[b][0])
        pltpu.async_copy(q_hbm.at[dst_v.at[c]], qbuf.at[b], sems[b][1])

    _fire(0, 0)
    _fire(1, 1)

    def _pair(i, _):
        for b in range(2):
            c = 2 * i + b
            pltpu.make_async_copy(p_hbm.at[src_v.at[c]], pbuf.at[b], sems[b][0]).wait()
            pltpu.make_async_copy(q_hbm.at[dst_v.at[c]], qbuf.at[b], sems[b][1]).wait()

            def _relu_row(r, _):
                for g in range(2):
                    s = pl.ds(g * 32, 32)
                    m = jnp.maximum(pbuf[b, r, s] + qbuf[b, r, s],
                                    jnp.zeros((32,), _bf16))
                    lo, hi = plsc.unpack(m, format=plsc.PackFormat.INTERLEAVED)
                    mbuf[b, r, pl.ds(g * 32, 16)] = lo
                    mbuf[b, r, pl.ds(g * 32 + 16, 16)] = hi
                return 0

            lax.fori_loop(0, C, _relu_row, 0)

            @pl.when(c + 2 < CPW)
            def _():
                _fire(c + 2, b)

            pltpu.sync_copy(mbuf.at[b], agg_sh.at[dst_v.at[c]], add=True)
        return 0

    lax.fori_loop(0, CPW // 2, _pair, 0)
    plsc.subcore_barrier()

    for k in range(RT // C):
        pltpu.sync_copy(agg_sh.at[pl.ds(row0 + k * C, C)], mbuf.at[0])
        pltpu.sync_copy(mbuf.at[0], out_hbm.at[cid, pl.ds(row0 + k * C, C)])


def kernel(time_segs, edges, W_edge, b_edge, W1, b1, W_d, b_d):
    x0 = time_segs[0, 0]
    x = jnp.zeros((NP_, ND), _f32).at[:NN].set(x0)

    wes = W_edge[:ND]
    wed = W_edge[ND:]
    be = b_edge.reshape(1, HE)
    w1a = W1[:ND]
    w1b = W1[ND:][_TAU]
    b1r = b1.reshape(1, HN)
    bdr = b_d.reshape(1, ND)

    pad = jnp.full((NEP - NE,), NN, jnp.int32)
    srcb = jnp.concatenate([edges[0], pad]).reshape(NW * CPW, C)
    dstb = jnp.concatenate([edges[1], pad]).reshape(NW * CPW, C)

    p, q = _pq_call(x, wes, wed, be)
    outs = []
    for step in range(PRED):
        aggp = _edge_agg(p, q, srcb, dstb)
        if step + 1 < PRED:
            x, p, q = _step_call(x, aggp, w1a, w1b, b1r, W_d, bdr, wes, wed, be)
        else:
            x = _final_call(x, aggp, w1a, w1b, b1r, W_d, bdr)
        outs.append(x[:NN])

    return jnp.stack(outs)[None]


# speedup vs baseline: 9.2379x; 1.0705x over previous
import functools

import numpy as np

import jax
import jax.numpy as jnp
from jax import lax
from jax.experimental import pallas as pl
from jax.experimental.pallas import tpu as pltpu
from jax.experimental.pallas import tpu_sc as plsc

NN = 10000
NE = 320000
ND = 128
HE = 64
HN = 64
PRED = 2

NP_ = 10240
RB = 1024
C = 128
NC, NS = 2, 16
NW = NC * NS
CPW = 80
NEP = NW * CPW * C
RT = NP_ // NS

_f32 = jnp.float32
_bf16 = jnp.bfloat16

_TAU = np.concatenate([np.arange(0, 32, 2), np.arange(1, 32, 2),
                       np.arange(32, 64, 2), np.arange(33, 64, 2)])


def _pq_body(x_ref, wes_ref, wed_ref, be_ref, p_ref, q_ref):
    x = x_ref[...]
    p_ref[...] = jnp.dot(x, wes_ref[...], preferred_element_type=_f32).astype(_bf16)
    q_ref[...] = (jnp.dot(x, wed_ref[...], preferred_element_type=_f32)
                  + be_ref[...]).astype(_bf16)


def _step_body(x_ref, agg_ref, w1a_ref, w1b_ref, b1_ref, wd_ref, bd_ref,
               wes_ref, wed_ref, be_ref, nxt_ref, p_ref, q_ref):
    x = x_ref[...]
    agg = agg_ref[0] + agg_ref[1]
    h = jnp.maximum(
        jnp.dot(x, w1a_ref[...], preferred_element_type=_f32)
        + jnp.dot(agg, w1b_ref[...], preferred_element_type=_f32)
        + b1_ref[...], 0.0)
    nxt = x + jnp.dot(h, wd_ref[...], preferred_element_type=_f32) + bd_ref[...]
    nxt_ref[...] = nxt
    p_ref[...] = jnp.dot(nxt, wes_ref[...], preferred_element_type=_f32).astype(_bf16)
    q_ref[...] = (jnp.dot(nxt, wed_ref[...], preferred_element_type=_f32)
                  + be_ref[...]).astype(_bf16)


def _final_body(x_ref, agg_ref, w1a_ref, w1b_ref, b1_ref, wd_ref, bd_ref,
                nxt_ref):
    x = x_ref[...]
    agg = agg_ref[0] + agg_ref[1]
    h = jnp.maximum(
        jnp.dot(x, w1a_ref[...], preferred_element_type=_f32)
        + jnp.dot(agg, w1b_ref[...], preferred_element_type=_f32)
        + b1_ref[...], 0.0)
    nxt_ref[...] = x + jnp.dot(h, wd_ref[...], preferred_element_type=_f32) + bd_ref[...]


def _full(shape):
    return pl.BlockSpec(shape, lambda i: tuple(0 for _ in shape))


def _rows(shape):
    return pl.BlockSpec(shape, lambda i: (i,) + tuple(0 for _ in shape[1:]))


_GRID = NP_ // RB

_pq_call = pl.pallas_call(
    _pq_body,
    grid=(_GRID,),
    in_specs=[_rows((RB, ND)), _full((ND, HE)), _full((ND, HE)), _full((1, HE))],
    out_specs=[_rows((RB, HE)), _rows((RB, HE))],
    out_shape=[jax.ShapeDtypeStruct((NP_, HE), _bf16)] * 2,
)

_step_call = pl.pallas_call(
    _step_body,
    grid=(_GRID,),
    in_specs=[
        _rows((RB, ND)),
        pl.BlockSpec((NC, RB, HE), lambda i: (0, i, 0)),
        _full((ND, HN)), _full((HE, HN)), _full((1, HN)),
        _full((HN, ND)), _full((1, ND)),
        _full((ND, HE)), _full((ND, HE)), _full((1, HE)),
    ],
    out_specs=[_rows((RB, ND)), _rows((RB, HE)), _rows((RB, HE))],
    out_shape=[jax.ShapeDtypeStruct((NP_, ND), _f32),
               jax.ShapeDtypeStruct((NP_, HE), _bf16),
               jax.ShapeDtypeStruct((NP_, HE), _bf16)],
)

_final_call = pl.pallas_call(
    _final_body,
    grid=(_GRID,),
    in_specs=[
        _rows((RB, ND)),
        pl.BlockSpec((NC, RB, HE), lambda i: (0, i, 0)),
        _full((ND, HN)), _full((HE, HN)), _full((1, HN)),
        _full((HN, ND)), _full((1, ND)),
    ],
    out_specs=_rows((RB, ND)),
    out_shape=jax.ShapeDtypeStruct((NP_, ND), _f32),
)


_mesh = plsc.VectorSubcoreMesh(core_axis_name="c", subcore_axis_name="s")


@functools.partial(
    pl.kernel,
    out_type=jax.ShapeDtypeStruct((NC, NP_, HE), _f32),
    mesh=_mesh,
    compiler_params=pltpu.CompilerParams(use_tc_tiling_on_sc=False,
                                         needs_layout_passes=False),
    scratch_types=[
        pltpu.VMEM((CPW, C), jnp.int32),
        pltpu.VMEM((CPW, C), jnp.int32),
        pltpu.VMEM((2, C, HE), _bf16),
        pltpu.VMEM((2, C, HE), _bf16),
        pltpu.VMEM((2, C, HE), _f32),
        pltpu.VMEM_SHARED((NP_, HE), _f32),
        pltpu.SemaphoreType.DMA,
        pltpu.SemaphoreType.DMA,
        pltpu.SemaphoreType.DMA,
        pltpu.SemaphoreType.DMA,
        pltpu.SemaphoreType.DMA,
        pltpu.SemaphoreType.DMA,
    ],
)
def _edge_agg(p_hbm, q_hbm, srcb_hbm, dstb_hbm, out_hbm,
              src_v, dst_v, pbuf, qbuf, mbuf, agg_sh,
              sp0, sp1, sq0, sq1, ss0, ss1):
    cid = lax.axis_index("c")
    sid = lax.axis_index("s")
    wid = sid * NC + cid
    row0 = sid * RT
    sems = ((sp0, sq0), (sp1, sq1))
    ssems = (ss0, ss1)

    def _zero_row(r, _):
        for c4 in range(HE // 16):
            mbuf[0, r, pl.ds(c4 * 16, 16)] = jnp.zeros((16,), _f32)
        return 0

    lax.fori_loop(0, C, _zero_row, 0)
    for k in range(RT // C):
        pltpu.sync_copy(mbuf.at[0], agg_sh.at[pl.ds(row0 + k * C, C)])

    pltpu.sync_copy(srcb_hbm.at[pl.ds(wid * CPW, CPW)], src_v)
    pltpu.sync_copy(dstb_hbm.at[pl.ds(wid * CPW, CPW)], dst_v)
    plsc.subcore_barrier()

    def _fire(c, b):
        pltpu.async_copy(p_hbm.at[src_v.at[c]], pbuf.at[b], sems[b][0])
        pltpu.async_copy(q_hbm.at[dst_v.at[c]], qbuf.at[b], sems[b][1])

    _fire(0, 0)
    _fire(1, 1)

    def _pair(i, _):
        for b in range(2):
            c = 2 * i + b
            pltpu.make_async_copy(p_hbm.at[src_v.at[c]], pbuf.at[b], sems[b][0]).wait()
            pltpu.make_async_copy(q_hbm.at[dst_v.at[c]], qbuf.at[b], sems[b][1]).wait()

            @pl.when(c >= 2)
            def _():
                pltpu.make_async_copy(
                    mbuf.at[b], agg_sh.at[dst_v.at[c]], ssems[b]).wait()

            def _relu_row(r, _):
                for g in range(2):
                    s = pl.ds(g * 32, 32)
                    m = jnp.maximum(pbuf[b, r, s] + qbuf[b, r, s],
                                    jnp.zeros((32,), _bf16))
                    lo, hi = plsc.unpack(m, format=plsc.PackFormat.INTERLEAVED)
                    mbuf[b, r, pl.ds(g * 32, 16)] = lo
                    mbuf[b, r, pl.ds(g * 32 + 16, 16)] = hi
                return 0

            lax.fori_loop(0, C, _relu_row, 0)

            @pl.when(c + 2 < CPW)
            def _():
                _fire(c + 2, b)

            pltpu.async_copy(mbuf.at[b], agg_sh.at[dst_v.at[c]], ssems[b],
                             add=True)
        return 0

    lax.fori_loop(0, CPW // 2, _pair, 0)
    for b in range(2):
        pltpu.make_async_copy(
            mbuf.at[b], agg_sh.at[dst_v.at[CPW - 2 + b]], ssems[b]).wait()
    plsc.subcore_barrier()

    for k in range(RT // C):
        pltpu.sync_copy(agg_sh.at[pl.ds(row0 + k * C, C)], mbuf.at[0])
        pltpu.sync_copy(mbuf.at[0], out_hbm.at[cid, pl.ds(row0 + k * C, C)])


def kernel(time_segs, edges, W_edge, b_edge, W1, b1, W_d, b_d):
    x0 = time_segs[0, 0]
    x = jnp.zeros((NP_, ND), _f32).at[:NN].set(x0)

    wes = W_edge[:ND]
    wed = W_edge[ND:]
    be = b_edge.reshape(1, HE)
    w1a = W1[:ND]
    w1b = W1[ND:][_TAU]
    b1r = b1.reshape(1, HN)
    bdr = b_d.reshape(1, ND)

    pad = jnp.full((NEP - NE,), NN, jnp.int32)
    srcb = jnp.concatenate([edges[0], pad]).reshape(NW * CPW, C)
    dstb = jnp.concatenate([edges[1], pad]).reshape(NW * CPW, C)

    p, q = _pq_call(x, wes, wed, be)
    outs = []
    for step in range(PRED):
        aggp = _edge_agg(p, q, srcb, dstb)
        if step + 1 < PRED:
            x, p, q = _step_call(x, aggp, w1a, w1b, b1r, W_d, bdr, wes, wed, be)
        else:
            x = _final_call(x, aggp, w1a, w1b, b1r, W_d, bdr)
        outs.append(x[:NN])

    return jnp.stack(outs)[None]
